# baseline (speedup 1.0000x reference)
import functools

import jax
import jax.numpy as jnp
from jax import lax
from jax.experimental import pallas as pl
from jax.experimental.pallas import tpu as pltpu
from jax.experimental.pallas import tpu_sc as plsc

_NV = 10000
_NC = 42000
_K = 3
_B = 128
_NSUB = 16
_ROWS = 2640
_ROWS_LAST = _NC - 15 * _ROWS
_CHUNK = 240
_GPC = _CHUNK // 16
_NCH = _ROWS // _CHUNK
_NCH_LAST = _ROWS_LAST // _CHUNK
_THRESH = float(_NC - 1)


def _sat_body(emb_hbm, idx_hbm, w_hbm, out_hbm,
              table_v, idx_s, w_s, part_v, part_sh, all_v, out_v,
              sem_i, sem_w):
    cid = lax.axis_index("c")
    sid = lax.axis_index("s")
    row0 = sid * _ROWS
    pltpu.sync_copy(emb_hbm.at[0], table_v)

    lanes = lax.iota(jnp.int32, 16)
    cols = [jnp.full((16,), j, jnp.int32) for j in range(_K)]
    two = jnp.full((16,), float(_K - 1), jnp.float32)

    def chunk_body(c, acc):
        r = pl.multiple_of(row0 + c * _CHUNK, 8)
        cp_i = pltpu.make_async_copy(idx_hbm.at[pl.ds(r, _CHUNK)], idx_s, sem_i)
        cp_w = pltpu.make_async_copy(w_hbm.at[pl.ds(r, _CHUNK)], w_s, sem_w)
        cp_i.start()
        cp_w.start()
        cp_i.wait()
        cp_w.wait()
        for g in range(_GPC):
            rows = lanes + g * 16
            pre = two
            for j in range(_K):
                lit = plsc.load_gather(idx_s, [rows, cols[j]])
                ev = plsc.load_gather(table_v, [lit])
                wv = plsc.load_gather(w_s, [rows, cols[j]])
                pre = pre + wv * jnp.sign(ev)
            acc = acc + jnp.sign(pre)
        return acc

    nchunks = jnp.where(sid == 15, _NCH_LAST, _NCH)
    acc = lax.fori_loop(0, nchunks, chunk_body, jnp.zeros((16,), jnp.float32))

    part_v[...] = acc
    pltpu.sync_copy(part_v, part_sh.at[sid])
    plsc.subcore_barrier()

    @pl.when(jnp.logical_and(cid == 0, sid == 0))
    def _finish():
        pltpu.sync_copy(part_sh, all_v)
        tot = all_v[0]
        for r in range(1, _NSUB):
            tot = tot + all_v[r]
        total = jnp.sum(tot)
        res = jnp.sign(total - _THRESH)
        resv = jnp.full((16,), res, jnp.float32)
        for k in range(_B // 16):
            out_v[pl.ds(k * 16, 16)] = resv
        pltpu.sync_copy(out_v, out_hbm)


@functools.lru_cache(maxsize=1)
def _build():
    mesh = plsc.VectorSubcoreMesh(
        core_axis_name="c", subcore_axis_name="s",
        num_cores=2, num_subcores=_NSUB,
    )
    return pl.kernel(
        _sat_body,
        out_type=jax.ShapeDtypeStruct((_B,), jnp.float32),
        mesh=mesh,
        compiler_params=pltpu.CompilerParams(needs_layout_passes=False),
        scratch_types=[
            pltpu.VMEM((_NV,), jnp.float32),
            pltpu.VMEM((_CHUNK, _K), jnp.int32),
            pltpu.VMEM((_CHUNK, _K), jnp.float32),
            pltpu.VMEM((16,), jnp.float32),
            pltpu.VMEM_SHARED((_NSUB, 16), jnp.float32),
            pltpu.VMEM((_NSUB, 16), jnp.float32),
            pltpu.VMEM((_B,), jnp.float32),
            pltpu.SemaphoreType.DMA,
            pltpu.SemaphoreType.DMA,
        ],
    )


def kernel(input, emb_weight, or_weight, clause_idx):
    del input
    return _build()(emb_weight, clause_idx, or_weight)
